# Initial kernel scaffold; baseline (speedup 1.0000x reference)
import jax
import jax.numpy as jnp
from jax.experimental import pallas as pl


def kernel(events):
    raise NotImplementedError("write your pallas kernel here")


# speedup vs baseline: 25.8826x; 25.8826x over previous
import functools

import jax
import jax.numpy as jnp
from jax import lax
from jax.experimental import pallas as pl
from jax.experimental.pallas import tpu as pltpu
from jax.experimental.pallas import tpu_sc as plsc

H, W = 260, 346
S, M = 16, 4
B = 8
NE = 2000000
NB = NE // B
HW = H * W
SHW = S * HW
ROWS = 128
LANES = 128
SUB = 32
PER_TILE = ROWS * LANES
NB_PAD = 16 * PER_TILE
BIN = SHW
GRID_W = SHW + 64
STRIPE = HW
NZ = 13
ZCH = STRIPE // NZ
RAND_IDXS = (12, 15, 6, 0)


def _prep_body(x_ref, y_ref, t_ref, o_ref):
    t = t_ref[0]
    tmax = jnp.max(t)
    ts = ((t / tmax) * float(S)).astype(jnp.int32)
    ts = jnp.where(ts == S, S - 1, ts)
    idx = x_ref[0].astype(jnp.int32) + W * y_ref[0].astype(jnp.int32) + HW * ts
    pad = jnp.full((1, NB_PAD - NB), BIN, jnp.int32)
    o_ref[0] = jnp.concatenate([idx, pad], axis=1)


def _make_scatter():
    mesh = plsc.VectorSubcoreMesh(core_axis_name="c", subcore_axis_name="s")

    @functools.partial(
        pl.kernel,
        mesh=mesh,
        out_type=jax.ShapeDtypeStruct((B * 16 * STRIPE,), jnp.float32),
        scratch_types=[
            pltpu.VMEM_SHARED((GRID_W,), jnp.float32),
            pltpu.VMEM((SUB, LANES), jnp.int32),
            pltpu.VMEM((LANES,), jnp.float32),
            pltpu.VMEM((ZCH,), jnp.float32),
            pltpu.VMEM((ZCH,), jnp.float32),
        ],
    )
    def scatter_k(idx_hbm, ones_hbm, zeros_hbm, out_hbm,
                  grid_sh, idx_v, ones_v, zero_v, bounce_v):
        c = lax.axis_index("c")
        s = lax.axis_index("s")
        pltpu.sync_copy(ones_hbm, ones_v)
        pltpu.sync_copy(zeros_hbm, zero_v)
        for k in range(B // 2):
            bi = c * (B // 2) + k
            for z in range(NZ):
                pltpu.sync_copy(zero_v,
                                grid_sh.at[pl.ds(s * STRIPE + z * ZCH, ZCH)])
            plsc.subcore_barrier()
            for sub in range(ROWS // SUB):
                pltpu.sync_copy(
                    idx_hbm.at[bi * 16 + s, pl.ds(sub * SUB, SUB)], idx_v)

                def _chunk(j, carry):
                    pltpu.sync_copy(ones_v, grid_sh.at[idx_v.at[j]], add=True)
                    return carry

                lax.fori_loop(0, SUB, _chunk, 0)
            plsc.subcore_barrier()
            for z in range(NZ):
                pltpu.sync_copy(grid_sh.at[pl.ds(s * STRIPE + z * ZCH, ZCH)],
                                bounce_v)
                pltpu.sync_copy(
                    bounce_v,
                    out_hbm.at[pl.ds((bi * 16 + s) * STRIPE + z * ZCH, ZCH)])

    return scatter_k


def _erode(g):
    g = g.astype(jnp.bfloat16).astype(jnp.float32)
    z_h = jnp.zeros((1, W), jnp.float32)
    z_w = jnp.zeros((H, 1), jnp.float32)
    up = jnp.concatenate([z_h, g[:-1, :]], axis=0)
    dn = jnp.concatenate([g[1:, :], z_h], axis=0)

    def xm(a):
        return jnp.concatenate([z_w, a[:, :-1]], axis=1)

    def xp(a):
        return jnp.concatenate([a[:, 1:], z_w], axis=1)

    acc = xm(up)
    for tap in (up, xp(up), xm(g), xp(g), xm(dn), dn, xp(dn)):
        acc = acc + tap
    return acc * 0.125 - 0.25


def _vox_body(c_ref, o_ref):
    counts = c_ref[0]
    dil = counts > 0.0
    dil_f = dil.astype(jnp.float32)
    mix_prev = dil_f[0]
    conc = dil_f[0]
    for i in range(1, S):
        a_i = 0.5 + (S - i) / float(S)
        b_i = i / float(S)
        g = dil_f[i] * a_i + mix_prev * b_i
        mi = _erode(g)
        conc = conc + (mi > 0.0).astype(jnp.float32)
        mix_prev = mi
    rl = conc > 0.0
    minreq = jnp.floor(jnp.sum(rl.astype(jnp.float32)) * (1.0 / S))
    for m, im in enumerate(RAND_IDXS):
        comb = jnp.logical_and(dil[im], rl)
        active = jnp.bool_(True)
        for cnt in range(1, S // M + 1):
            scomb = jnp.sum(comb.astype(jnp.float32))
            active = jnp.logical_and(active, scomb < minreq)
            j = im + cnt if im < M // 2 else im - cnt
            add = jnp.logical_and(dil[j], rl)
            comb = jnp.logical_or(comb, jnp.logical_and(add, active))
        o_ref[0, 1 + m] = comb.astype(jnp.float32)
    o_ref[0, 0] = conc / jnp.max(conc)


def kernel(events):
    x = events[:, 0].reshape(B, 1, NB)
    y = events[:, 1].reshape(B, 1, NB)
    t = events[:, 2].reshape(B, 1, NB)
    idxp = pl.pallas_call(
        _prep_body,
        grid=(B,),
        in_specs=[pl.BlockSpec((1, 1, NB), lambda i: (i, 0, 0))] * 3,
        out_specs=pl.BlockSpec((1, 1, NB_PAD), lambda i: (i, 0, 0)),
        out_shape=jax.ShapeDtypeStruct((B, 1, NB_PAD), jnp.int32),
    )(x, y, t)
    idx4 = idxp.reshape(B * 16, ROWS, LANES)
    ones_in = jnp.ones((LANES,), jnp.float32)
    zeros_in = jnp.zeros((ZCH,), jnp.float32)
    counts = _make_scatter()(idx4, ones_in, zeros_in)
    counts4 = counts.reshape(B, S, H, W)
    vox = pl.pallas_call(
        _vox_body,
        grid=(B,),
        in_specs=[pl.BlockSpec((1, S, H, W), lambda i: (i, 0, 0, 0))],
        out_specs=pl.BlockSpec((1, M + 1, H, W), lambda i: (i, 0, 0, 0)),
        out_shape=jax.ShapeDtypeStruct((B, M + 1, H, W), jnp.float32),
    )(counts4)
    return vox
